# baseline (speedup 1.0000x reference)
import jax
import jax.numpy as jnp
from jax import lax
from jax.experimental import pallas as pl
from jax.experimental.pallas import tpu as pltpu
from jax.experimental.pallas import tpu_sc as plsc

BATCH = 4096
MAX_LEN = 200
EMB_DIM = 64
OUT_DIM = 8

L = 16
G0 = 112
G1 = 88
NW = 32
BPW = BATCH // NW


def _sc_body(idxflat, table, out_hbm, idxblk, rows0, rows1, outbuf, sem):
    nc = 2
    wid = lax.axis_index("s") * nc + lax.axis_index("c")
    base = wid * BPW

    pltpu.sync_copy(idxflat.at[pl.ds(base * MAX_LEN, BPW * MAX_LEN)], idxblk)

    def per_row(b, _):
        off = pl.multiple_of(b * MAX_LEN, 8)
        cp0 = pltpu.async_copy(table.at[idxblk.at[pl.ds(off, G0)]], rows0, sem)
        cp1 = pltpu.async_copy(table.at[idxblk.at[pl.ds(off + G0, G1)]], rows1, sem)
        cp0.wait()
        cp1.wait()

        def acc2(j, acc):
            a0, a1, a2, a3 = acc
            a0 = a0 + rows0[j, pl.ds(0 * L, L)] + rows1[j, pl.ds(0 * L, L)]
            a1 = a1 + rows0[j, pl.ds(1 * L, L)] + rows1[j, pl.ds(1 * L, L)]
            a2 = a2 + rows0[j, pl.ds(2 * L, L)] + rows1[j, pl.ds(2 * L, L)]
            a3 = a3 + rows0[j, pl.ds(3 * L, L)] + rows1[j, pl.ds(3 * L, L)]
            return (a0, a1, a2, a3)

        def acc1(j, acc):
            a0, a1, a2, a3 = acc
            a0 = a0 + rows0[j, pl.ds(0 * L, L)]
            a1 = a1 + rows0[j, pl.ds(1 * L, L)]
            a2 = a2 + rows0[j, pl.ds(2 * L, L)]
            a3 = a3 + rows0[j, pl.ds(3 * L, L)]
            return (a0, a1, a2, a3)

        zf = jnp.zeros((L,), jnp.float32)
        acc = lax.fori_loop(0, G1, acc2, (zf, zf, zf, zf))
        acc = lax.fori_loop(G1, G0, acc1, acc)
        for d in range(4):
            outbuf[pl.ds(b * EMB_DIM + d * L, L)] = acc[d]
        return _

    lax.fori_loop(0, BPW, per_row, None)
    pltpu.sync_copy(outbuf, out_hbm.at[pl.ds(base * EMB_DIM, BPW * EMB_DIM)])


def _sc_gather_sum(idxflat, table):
    mesh = plsc.VectorSubcoreMesh(core_axis_name="c", subcore_axis_name="s")
    return pl.kernel(
        _sc_body,
        mesh=mesh,
        compiler_params=pltpu.CompilerParams(use_tc_tiling_on_sc=False),
        out_type=jax.ShapeDtypeStruct((BATCH * EMB_DIM,), jnp.float32),
        scratch_types=[
            pltpu.VMEM((BPW * MAX_LEN,), jnp.int32),
            pltpu.VMEM((G0, EMB_DIM), jnp.float32),
            pltpu.VMEM((G1, EMB_DIM), jnp.float32),
            pltpu.VMEM((BPW * EMB_DIM,), jnp.float32),
            pltpu.SemaphoreType.DMA,
        ],
    )(idxflat, table)


def _finish_body(s_ref, d_ref, r0_ref, l_ref, w_ref, b_ref, o_ref):
    cnt = jnp.sum(jnp.where(d_ref[:] == 0, 1.0, 0.0), axis=1, keepdims=True)
    x = (s_ref[:] - cnt * r0_ref[:]) / l_ref[:]
    o_ref[:] = jnp.dot(x, w_ref[:], preferred_element_type=jnp.float32) + b_ref[:]


def _tc_finish(sums, data_i, row0, lenf, w1t, b1r):
    return pl.pallas_call(
        _finish_body,
        out_shape=jax.ShapeDtypeStruct((BATCH, OUT_DIM), jnp.float32),
    )(sums, data_i, row0, lenf, w1t, b1r)


def kernel(data, length, embed_table, W1, b1):
    data_i = data.astype(jnp.int32)
    sums = _sc_gather_sum(data_i.reshape(-1), embed_table).reshape(BATCH, EMB_DIM)
    lenf = length.astype(jnp.float32).reshape(BATCH, 1)
    return _tc_finish(sums, data_i, embed_table[0:1], lenf, W1.T,
                      b1.reshape(1, OUT_DIM))


# speedup vs baseline: 1.1361x; 1.1361x over previous
import jax
import jax.numpy as jnp
from jax import lax
from jax.experimental import pallas as pl
from jax.experimental.pallas import tpu as pltpu
from jax.experimental.pallas import tpu_sc as plsc

BATCH = 4096
MAX_LEN = 200
EMB_DIM = 64
OUT_DIM = 8

L = 16
G0 = 112
G1 = 88
NW = 32
BPW = BATCH // NW


def _sc_body(idxflat, table, out_hbm, idxblk,
             rows0a, rows1a, rows0b, rows1b, outbuf, sema, semb):
    nc = 2
    wid = lax.axis_index("s") * nc + lax.axis_index("c")
    base = wid * BPW

    pltpu.sync_copy(idxflat.at[pl.ds(base * MAX_LEN, BPW * MAX_LEN)], idxblk)

    def issue(b, r0, r1, sem):
        off = pl.multiple_of(b * MAX_LEN, 8)
        pltpu.async_copy(table.at[idxblk.at[pl.ds(off, G0)]], r0, sem)
        pltpu.async_copy(table.at[idxblk.at[pl.ds(off + G0, G1)]], r1, sem)

    def wait(r0, r1, sem):
        pltpu.make_async_copy(table.at[idxblk.at[pl.ds(0, G0)]], r0, sem).wait()
        pltpu.make_async_copy(table.at[idxblk.at[pl.ds(0, G1)]], r1, sem).wait()

    def consume(b, r0, r1):
        def acc2(j, acc):
            a0, a1, a2, a3 = acc
            a0 = a0 + r0[j, pl.ds(0 * L, L)] + r1[j, pl.ds(0 * L, L)]
            a1 = a1 + r0[j, pl.ds(1 * L, L)] + r1[j, pl.ds(1 * L, L)]
            a2 = a2 + r0[j, pl.ds(2 * L, L)] + r1[j, pl.ds(2 * L, L)]
            a3 = a3 + r0[j, pl.ds(3 * L, L)] + r1[j, pl.ds(3 * L, L)]
            return (a0, a1, a2, a3)

        def acc1(j, acc):
            a0, a1, a2, a3 = acc
            a0 = a0 + r0[j, pl.ds(0 * L, L)]
            a1 = a1 + r0[j, pl.ds(1 * L, L)]
            a2 = a2 + r0[j, pl.ds(2 * L, L)]
            a3 = a3 + r0[j, pl.ds(3 * L, L)]
            return (a0, a1, a2, a3)

        zf = jnp.zeros((L,), jnp.float32)
        acc = lax.fori_loop(0, G1, acc2, (zf, zf, zf, zf))
        acc = lax.fori_loop(G1, G0, acc1, acc)
        for d in range(4):
            outbuf[pl.ds(b * EMB_DIM + d * L, L)] = acc[d]

    issue(0, rows0a, rows1a, sema)

    def per_pair(g, _):
        b0 = pl.multiple_of(g * 2, 2)
        issue(b0 + 1, rows0b, rows1b, semb)
        wait(rows0a, rows1a, sema)
        consume(b0, rows0a, rows1a)

        @pl.when(g < BPW // 2 - 1)
        def _issue_next():
            issue(b0 + 2, rows0a, rows1a, sema)

        wait(rows0b, rows1b, semb)
        consume(b0 + 1, rows0b, rows1b)
        return _

    lax.fori_loop(0, BPW // 2, per_pair, None)
    pltpu.sync_copy(outbuf, out_hbm.at[pl.ds(base * EMB_DIM, BPW * EMB_DIM)])


def _sc_gather_sum(idxflat, table):
    mesh = plsc.VectorSubcoreMesh(core_axis_name="c", subcore_axis_name="s")
    return pl.kernel(
        _sc_body,
        mesh=mesh,
        compiler_params=pltpu.CompilerParams(use_tc_tiling_on_sc=False),
        out_type=jax.ShapeDtypeStruct((BATCH * EMB_DIM,), jnp.float32),
        scratch_types=[
            pltpu.VMEM((BPW * MAX_LEN,), jnp.int32),
            pltpu.VMEM((G0, EMB_DIM), jnp.float32),
            pltpu.VMEM((G1, EMB_DIM), jnp.float32),
            pltpu.VMEM((G0, EMB_DIM), jnp.float32),
            pltpu.VMEM((G1, EMB_DIM), jnp.float32),
            pltpu.VMEM((BPW * EMB_DIM,), jnp.float32),
            pltpu.SemaphoreType.DMA,
            pltpu.SemaphoreType.DMA,
        ],
    )(idxflat, table)


def _finish_body(s_ref, d_ref, r0_ref, l_ref, w_ref, b_ref, o_ref):
    cnt = jnp.sum(jnp.where(d_ref[:] == 0, 1.0, 0.0), axis=1, keepdims=True)
    x = (s_ref[:] - cnt * r0_ref[:]) / l_ref[:]
    o_ref[:] = jnp.dot(x, w_ref[:], preferred_element_type=jnp.float32) + b_ref[:]


def _tc_finish(sums, data_i, row0, lenf, w1t, b1r):
    return pl.pallas_call(
        _finish_body,
        out_shape=jax.ShapeDtypeStruct((BATCH, OUT_DIM), jnp.float32),
    )(sums, data_i, row0, lenf, w1t, b1r)


def kernel(data, length, embed_table, W1, b1):
    data_i = data.astype(jnp.int32)
    sums = _sc_gather_sum(data_i.reshape(-1), embed_table).reshape(BATCH, EMB_DIM)
    lenf = length.astype(jnp.float32).reshape(BATCH, 1)
    return _tc_finish(sums, data_i, embed_table[0:1], lenf, W1.T,
                      b1.reshape(1, OUT_DIM))
